# baseline (speedup 1.0000x reference)
import functools

import jax
import jax.numpy as jnp
from jax import lax
from jax.experimental import pallas as pl
from jax.experimental.pallas import tpu as pltpu
from jax.experimental.pallas import tpu_sc as plsc

SIGMA = 1.0
NUM_CORES = 2
NUM_SUBCORES = 16
NUM_WORKERS = NUM_CORES * NUM_SUBCORES
LANES = 16
FBLK = 128
NBUF = 7
PAD = 16


def _body(latent_dim, b_per_w, style_hbm, frame_hbm, lat_hbm, mu_hbm,
          out_hbm, sid_s, fid_s, slabring, murring, out_v, sems):
    wid = lax.axis_index("s") * NUM_CORES + lax.axis_index("c")
    base = wid * b_per_w
    lag = NBUF - 1

    pltpu.sync_copy(style_hbm.at[pl.ds(base, b_per_w)],
                    sid_s.at[pl.ds(PAD, b_per_w)])
    pltpu.sync_copy(frame_hbm.at[pl.ds(base, b_per_w)],
                    fid_s.at[pl.ds(PAD, b_per_w)])

    def sid(b):
        return sid_s[pl.ds(b + PAD, LANES)][0]

    def fid(b):
        return fid_s[pl.ds(b + PAD, LANES)][0]

    def key(b):
        return sid(b) * 8 + fid(b) // FBLK

    def slab_src(b):
        off = pl.multiple_of((fid(b) // FBLK) * FBLK, FBLK)
        return lat_hbm.at[sid(b), :, pl.ds(off, FBLK)]

    def mu_src(b):
        return mu_hbm.at[sid(b), :]

    d_iota = lax.broadcasted_iota(jnp.int32, (LANES,), 0)
    one = jnp.int32(1)
    zero = jnp.int32(0)

    def step(j, carry):
        nf, nw = carry

        new_f = jnp.where(
            j < b_per_w,
            jnp.where(j == 0, one,
                      (key(j) != key(j - 1)).astype(jnp.int32)),
            zero)

        @pl.when(new_f == 1)
        def _fire():
            for k in range(NBUF):
                @pl.when(nf % NBUF == k)
                def _(k=k):
                    pltpu.async_copy(slab_src(j), slabring.at[k], sems[k])
                    pltpu.async_copy(mu_src(j), murring.at[k], sems[k])

        e = j - lag
        new_e = jnp.where(
            j >= lag,
            jnp.where(e == 0, one,
                      (key(e) != key(e - 1)).astype(jnp.int32)),
            zero)
        nw2 = nw + new_e
        slot = (nw2 - 1) % NBUF

        @pl.when(j >= lag)
        def _consume():
            @pl.when(new_e == 1)
            def _wait():
                for k in range(NBUF):
                    @pl.when(slot == k)
                    def _(k=k):
                        pltpu.make_async_copy(slab_src(e), slabring.at[k],
                                              sems[k]).wait()
                        pltpu.make_async_copy(mu_src(e), murring.at[k],
                                              sems[k]).wait()

            foff = fid(e) % FBLK
            col_idx = jnp.full((LANES,), foff, jnp.int32)
            slot_v = jnp.full((LANES,), slot, jnp.int32)
            for d0 in range(0, latent_dim, LANES):
                lat = plsc.load_gather(slabring,
                                       [slot_v, d0 + d_iota, col_idx])
                mu = plsc.load_gather(murring, [slot_v, d0 + d_iota])
                out_v[e, pl.ds(d0, LANES)] = mu + SIGMA * (lat - mu)

        return nf + new_f, nw2

    lax.fori_loop(0, b_per_w + lag, step, (zero, zero))

    pltpu.sync_copy(out_v, out_hbm.at[pl.ds(base, b_per_w)])


def kernel(style_ids, frame_ids, latents, style_latents_mu):
    style_num, frame_num, latent_dim = latents.shape
    batch = style_ids.shape[0]
    b_per_w = batch // NUM_WORKERS
    lat_t = jnp.transpose(latents, (0, 2, 1))

    key = style_ids * 8 + frame_ids // FBLK
    packed = key * batch + jnp.arange(batch, dtype=jnp.int32)
    order = jnp.sort(packed) % batch
    style_s = style_ids[order]
    frame_s = frame_ids[order]

    mesh = plsc.VectorSubcoreMesh(core_axis_name="c", subcore_axis_name="s",
                                  num_cores=NUM_CORES,
                                  num_subcores=NUM_SUBCORES)
    run = pl.kernel(
        functools.partial(_body, latent_dim, b_per_w),
        out_type=jax.ShapeDtypeStruct((batch, latent_dim), jnp.float32),
        mesh=mesh,
        scratch_types=[
            pltpu.VMEM((b_per_w + PAD + 64,), jnp.int32),
            pltpu.VMEM((b_per_w + PAD + 64,), jnp.int32),
            pltpu.VMEM((NBUF, latent_dim, FBLK), jnp.float32),
            pltpu.VMEM((NBUF, latent_dim), jnp.float32),
            pltpu.VMEM((b_per_w, latent_dim), jnp.float32),
            [pltpu.SemaphoreType.DMA for _ in range(NBUF)],
        ],
        compiler_params=pltpu.CompilerParams(needs_layout_passes=False),
    )
    res = run(style_s, frame_s, lat_t, style_latents_mu)
    inv = jnp.zeros_like(order).at[order].set(
        jnp.arange(batch, dtype=order.dtype))
    return res[inv]


# speedup vs baseline: 1.0830x; 1.0830x over previous
import functools

import jax
import jax.numpy as jnp
from jax import lax
from jax.experimental import pallas as pl
from jax.experimental.pallas import tpu as pltpu
from jax.experimental.pallas import tpu_sc as plsc

SIGMA = 1.0
NUM_CORES = 2
NUM_SUBCORES = 16
NUM_WORKERS = NUM_CORES * NUM_SUBCORES
LANES = 16
FBLK = 128
NBUF = 7
PAD = 16


def _body(latent_dim, b_per_w, kf_hbm, lat_hbm, mu_hbm,
          out_hbm, kf_s, slabring, murring, out_v, sems):
    wid = lax.axis_index("s") * NUM_CORES + lax.axis_index("c")
    base = wid * b_per_w
    lag = NBUF - 1

    pltpu.sync_copy(kf_hbm.at[pl.ds(base, b_per_w)],
                    kf_s.at[pl.ds(PAD, b_per_w)])

    def kf(b):
        return kf_s[pl.ds(b + PAD, LANES)][0]

    def sid(b):
        return kf(b) >> 10

    def fid(b):
        return kf(b) & 1023

    def key(b):
        return kf(b) >> 7

    def slab_src(b):
        off = pl.multiple_of((fid(b) // FBLK) * FBLK, FBLK)
        return lat_hbm.at[sid(b), :, pl.ds(off, FBLK)]

    def mu_src(b):
        return mu_hbm.at[sid(b), :]

    d_iota = lax.broadcasted_iota(jnp.int32, (LANES,), 0)
    one = jnp.int32(1)
    zero = jnp.int32(0)

    def step(j, carry):
        nf, nw = carry

        new_f = jnp.where(
            j < b_per_w,
            jnp.where(j == 0, one,
                      (key(j) != key(j - 1)).astype(jnp.int32)),
            zero)

        @pl.when(new_f == 1)
        def _fire():
            for k in range(NBUF):
                @pl.when(nf % NBUF == k)
                def _(k=k):
                    pltpu.async_copy(slab_src(j), slabring.at[k], sems[k])
                    pltpu.async_copy(mu_src(j), murring.at[k], sems[k])

        e = j - lag
        new_e = jnp.where(
            j >= lag,
            jnp.where(e == 0, one,
                      (key(e) != key(e - 1)).astype(jnp.int32)),
            zero)
        nw2 = nw + new_e
        slot = (nw2 - 1) % NBUF

        @pl.when(j >= lag)
        def _consume():
            @pl.when(new_e == 1)
            def _wait():
                for k in range(NBUF):
                    @pl.when(slot == k)
                    def _(k=k):
                        pltpu.make_async_copy(slab_src(e), slabring.at[k],
                                              sems[k]).wait()
                        pltpu.make_async_copy(mu_src(e), murring.at[k],
                                              sems[k]).wait()

            foff = fid(e) % FBLK
            col_idx = jnp.full((LANES,), foff, jnp.int32)
            slot_v = jnp.full((LANES,), slot, jnp.int32)
            for d0 in range(0, latent_dim, LANES):
                lat = plsc.load_gather(slabring,
                                       [slot_v, d0 + d_iota, col_idx])
                mu = murring[slot, pl.ds(d0, LANES)]
                out_v[e, pl.ds(d0, LANES)] = mu + SIGMA * (lat - mu)

        return nf + new_f, nw2

    lax.fori_loop(0, b_per_w + lag, step, (zero, zero))

    pltpu.sync_copy(out_v, out_hbm.at[pl.ds(base, b_per_w)])


def kernel(style_ids, frame_ids, latents, style_latents_mu):
    style_num, frame_num, latent_dim = latents.shape
    batch = style_ids.shape[0]
    b_per_w = batch // NUM_WORKERS
    lat_t = jnp.transpose(latents, (0, 2, 1))

    kf = style_ids * 1024 + frame_ids
    kf_sorted, order = lax.sort_key_val(
        kf, jnp.arange(batch, dtype=jnp.int32))

    mesh = plsc.VectorSubcoreMesh(core_axis_name="c", subcore_axis_name="s",
                                  num_cores=NUM_CORES,
                                  num_subcores=NUM_SUBCORES)
    run = pl.kernel(
        functools.partial(_body, latent_dim, b_per_w),
        out_type=jax.ShapeDtypeStruct((batch, latent_dim), jnp.float32),
        mesh=mesh,
        scratch_types=[
            pltpu.VMEM((b_per_w + PAD + 64,), jnp.int32),
            pltpu.VMEM((NBUF, latent_dim, FBLK), jnp.float32),
            pltpu.VMEM((NBUF, latent_dim), jnp.float32),
            pltpu.VMEM((b_per_w, latent_dim), jnp.float32),
            [pltpu.SemaphoreType.DMA for _ in range(NBUF)],
        ],
        compiler_params=pltpu.CompilerParams(needs_layout_passes=False),
    )
    res = run(kf_sorted, lat_t, style_latents_mu)
    inv = jnp.zeros_like(order).at[order].set(
        jnp.arange(batch, dtype=order.dtype))
    return res[inv]


# speedup vs baseline: 1.1562x; 1.0676x over previous
import functools

import jax
import jax.numpy as jnp
from jax import lax
from jax.experimental import pallas as pl
from jax.experimental.pallas import tpu as pltpu
from jax.experimental.pallas import tpu_sc as plsc

SIGMA = 1.0
NUM_CORES = 2
NUM_SUBCORES = 16
NUM_WORKERS = NUM_CORES * NUM_SUBCORES
LANES = 16
FBLK = 128
NBUF = 9
PAD = 16


def _body(latent_dim, b_per_w, kf_hbm, lat_hbm, mu_hbm,
          out_hbm, kf_s, slabring, murring, out_v, sems):
    wid = lax.axis_index("s") * NUM_CORES + lax.axis_index("c")
    base = wid * b_per_w
    lag = NBUF - 1

    pltpu.sync_copy(kf_hbm.at[pl.ds(base, b_per_w)],
                    kf_s.at[pl.ds(PAD, b_per_w)])

    def kf(b):
        return kf_s[pl.ds(b + PAD, LANES)][0]

    def sid(b):
        return kf(b) >> 10

    def fid(b):
        return kf(b) & 1023

    def key(b):
        return kf(b) >> 7

    def slab_src(b):
        off = pl.multiple_of((fid(b) // FBLK) * FBLK, FBLK)
        return lat_hbm.at[sid(b), :, pl.ds(off, FBLK)]

    def mu_src(b):
        return mu_hbm.at[sid(b), :]

    d_iota = lax.broadcasted_iota(jnp.int32, (LANES,), 0)
    one = jnp.int32(1)
    zero = jnp.int32(0)

    def step(j, carry):
        nf, nw = carry

        new_f = jnp.where(
            j < b_per_w,
            jnp.where(j == 0, one,
                      (key(j) != key(j - 1)).astype(jnp.int32)),
            zero)

        @pl.when(new_f == 1)
        def _fire():
            for k in range(NBUF):
                @pl.when(nf % NBUF == k)
                def _(k=k):
                    pltpu.async_copy(slab_src(j), slabring.at[k], sems[k])
                    pltpu.async_copy(mu_src(j), murring.at[k], sems[k])

        e = j - lag
        new_e = jnp.where(
            j >= lag,
            jnp.where(e == 0, one,
                      (key(e) != key(e - 1)).astype(jnp.int32)),
            zero)
        nw2 = nw + new_e
        slot = (nw2 - 1) % NBUF

        @pl.when(j >= lag)
        def _consume():
            @pl.when(e == b_per_w // 2)
            def _flush():
                pltpu.sync_copy(out_v,
                                out_hbm.at[pl.ds(base, b_per_w // 2)])

            @pl.when(new_e == 1)
            def _wait():
                for k in range(NBUF):
                    @pl.when(slot == k)
                    def _(k=k):
                        pltpu.make_async_copy(slab_src(e), slabring.at[k],
                                              sems[k]).wait()
                        pltpu.make_async_copy(mu_src(e), murring.at[k],
                                              sems[k]).wait()

            foff = fid(e) % FBLK
            col_idx = jnp.full((LANES,), foff, jnp.int32)
            slot_v = jnp.full((LANES,), slot, jnp.int32)
            for d0 in range(0, latent_dim, LANES):
                lat = plsc.load_gather(slabring,
                                       [slot_v, d0 + d_iota, col_idx])
                mu = murring[slot, pl.ds(d0, LANES)]
                out_v[e % (b_per_w // 2), pl.ds(d0, LANES)] = (
                    mu + SIGMA * (lat - mu))

        return nf + new_f, nw2

    lax.fori_loop(0, b_per_w + lag, step, (zero, zero))

    pltpu.sync_copy(out_v,
                    out_hbm.at[pl.ds(base + b_per_w // 2, b_per_w // 2)])


def kernel(style_ids, frame_ids, latents, style_latents_mu):
    style_num, frame_num, latent_dim = latents.shape
    batch = style_ids.shape[0]
    b_per_w = batch // NUM_WORKERS
    lat_t = jnp.transpose(latents, (0, 2, 1))

    kf = style_ids * 1024 + frame_ids
    kf_sorted, order = lax.sort_key_val(
        kf, jnp.arange(batch, dtype=jnp.int32))

    mesh = plsc.VectorSubcoreMesh(core_axis_name="c", subcore_axis_name="s",
                                  num_cores=NUM_CORES,
                                  num_subcores=NUM_SUBCORES)
    run = pl.kernel(
        functools.partial(_body, latent_dim, b_per_w),
        out_type=jax.ShapeDtypeStruct((batch, latent_dim), jnp.float32),
        mesh=mesh,
        scratch_types=[
            pltpu.VMEM((b_per_w + PAD + 64,), jnp.int32),
            pltpu.VMEM((NBUF, latent_dim, FBLK), jnp.float32),
            pltpu.VMEM((NBUF, latent_dim), jnp.float32),
            pltpu.VMEM((b_per_w // 2, latent_dim), jnp.float32),
            [pltpu.SemaphoreType.DMA for _ in range(NBUF)],
        ],
        compiler_params=pltpu.CompilerParams(needs_layout_passes=False),
    )
    res = run(kf_sorted, lat_t, style_latents_mu)
    inv = jnp.zeros_like(order).at[order].set(
        jnp.arange(batch, dtype=order.dtype))
    return res[inv]


# speedup vs baseline: 1.1744x; 1.0157x over previous
import functools

import jax
import jax.numpy as jnp
from jax import lax
from jax.experimental import pallas as pl
from jax.experimental.pallas import tpu as pltpu
from jax.experimental.pallas import tpu_sc as plsc

SIGMA = 1.0
NUM_CORES = 2
NUM_SUBCORES = 16
NUM_WORKERS = NUM_CORES * NUM_SUBCORES
LANES = 16
FBLK = 128
NBUF = 11
RN = 8
PAD = 16


def _body(latent_dim, b_per_w, kf_hbm, ord_hbm, lat_hbm, mu_hbm,
          out_hbm, kf_s, ord_s, slabring, murring, res, sems, osems):
    wid = lax.axis_index("s") * NUM_CORES + lax.axis_index("c")
    base = wid * b_per_w
    lag = NBUF - 1

    pltpu.sync_copy(kf_hbm.at[pl.ds(base, b_per_w)],
                    kf_s.at[pl.ds(PAD, b_per_w)])
    pltpu.sync_copy(ord_hbm.at[pl.ds(base, b_per_w)],
                    ord_s.at[pl.ds(PAD, b_per_w)])

    def kf(b):
        return kf_s[pl.ds(b + PAD, LANES)][0]

    def ordv(b):
        return ord_s[pl.ds(b + PAD, LANES)][0]

    def sid(b):
        return kf(b) >> 10

    def fid(b):
        return kf(b) & 1023

    def key(b):
        return kf(b) >> 7

    def slab_src(b):
        off = pl.multiple_of((fid(b) // FBLK) * FBLK, FBLK)
        return lat_hbm.at[sid(b), :, pl.ds(off, FBLK)]

    def mu_src(b):
        return mu_hbm.at[sid(b), :]

    d_iota = lax.broadcasted_iota(jnp.int32, (LANES,), 0)
    one = jnp.int32(1)
    zero = jnp.int32(0)

    def step(j, carry):
        nf, nw = carry

        new_f = jnp.where(
            j < b_per_w,
            jnp.where(j == 0, one,
                      (key(j) != key(j - 1)).astype(jnp.int32)),
            zero)

        @pl.when(new_f == 1)
        def _fire():
            for k in range(NBUF):
                @pl.when(nf % NBUF == k)
                def _(k=k):
                    pltpu.async_copy(slab_src(j), slabring.at[k], sems[k])
                    pltpu.async_copy(mu_src(j), murring.at[k], sems[k])

        e = j - lag
        new_e = jnp.where(
            j >= lag,
            jnp.where(e == 0, one,
                      (key(e) != key(e - 1)).astype(jnp.int32)),
            zero)
        nw2 = nw + new_e
        slot = (nw2 - 1) % NBUF

        @pl.when(j >= lag)
        def _consume():
            @pl.when(new_e == 1)
            def _wait():
                for k in range(NBUF):
                    @pl.when(slot == k)
                    def _(k=k):
                        pltpu.make_async_copy(slab_src(e), slabring.at[k],
                                              sems[k]).wait()
                        pltpu.make_async_copy(mu_src(e), murring.at[k],
                                              sems[k]).wait()

            @pl.when(e >= RN)
            def _retire():
                for r in range(RN):
                    @pl.when(e % RN == r)
                    def _(r=r):
                        pltpu.make_async_copy(
                            res.at[r], out_hbm.at[ordv(e - RN), :],
                            osems[r]).wait()

            foff = fid(e) % FBLK
            col_idx = jnp.full((LANES,), foff, jnp.int32)
            slot_v = jnp.full((LANES,), slot, jnp.int32)
            for d0 in range(0, latent_dim, LANES):
                lat = plsc.load_gather(slabring,
                                       [slot_v, d0 + d_iota, col_idx])
                mu = murring[slot, pl.ds(d0, LANES)]
                res[e % RN, pl.ds(d0, LANES)] = mu + SIGMA * (lat - mu)

            for r in range(RN):
                @pl.when(e % RN == r)
                def _(r=r):
                    pltpu.async_copy(res.at[r], out_hbm.at[ordv(e), :],
                                     osems[r])

        return nf + new_f, nw2

    lax.fori_loop(0, b_per_w + lag, step, (zero, zero))

    def drain(e, _):
        for r in range(RN):
            @pl.when(e % RN == r)
            def _(r=r):
                pltpu.make_async_copy(res.at[r], out_hbm.at[ordv(e), :],
                                      osems[r]).wait()
        return 0

    lax.fori_loop(b_per_w - RN, b_per_w, drain, 0)


def kernel(style_ids, frame_ids, latents, style_latents_mu):
    style_num, frame_num, latent_dim = latents.shape
    batch = style_ids.shape[0]
    b_per_w = batch // NUM_WORKERS
    lat_t = jnp.transpose(latents, (0, 2, 1))

    kf = style_ids * 1024 + frame_ids
    kf_sorted, order = lax.sort_key_val(
        kf, jnp.arange(batch, dtype=jnp.int32))

    mesh = plsc.VectorSubcoreMesh(core_axis_name="c", subcore_axis_name="s",
                                  num_cores=NUM_CORES,
                                  num_subcores=NUM_SUBCORES)
    run = pl.kernel(
        functools.partial(_body, latent_dim, b_per_w),
        out_type=jax.ShapeDtypeStruct((batch, latent_dim), jnp.float32),
        mesh=mesh,
        scratch_types=[
            pltpu.VMEM((b_per_w + PAD + 64,), jnp.int32),
            pltpu.VMEM((b_per_w + PAD + 64,), jnp.int32),
            pltpu.VMEM((NBUF, latent_dim, FBLK), jnp.float32),
            pltpu.VMEM((NBUF, latent_dim), jnp.float32),
            pltpu.VMEM((RN, latent_dim), jnp.float32),
            [pltpu.SemaphoreType.DMA for _ in range(NBUF)],
            [pltpu.SemaphoreType.DMA for _ in range(RN)],
        ],
        compiler_params=pltpu.CompilerParams(needs_layout_passes=False),
    )
    return run(kf_sorted, order, lat_t, style_latents_mu)


# speedup vs baseline: 1.2884x; 1.0971x over previous
import functools

import jax
import jax.numpy as jnp
from jax import lax
from jax.experimental import pallas as pl
from jax.experimental.pallas import tpu as pltpu
from jax.experimental.pallas import tpu_sc as plsc

SIGMA = 1.0
NUM_CORES = 2
NUM_SUBCORES = 16
NUM_WORKERS = NUM_CORES * NUM_SUBCORES
LANES = 16
FBLK = 128
NBUF = 12
RN = 4
PAD = 16


def _body(latent_dim, b_per_w, kf_hbm, ord_hbm, lat_hbm, mu_hbm,
          out_hbm, kf_s, ord_s, slabring, murring, res, sems, osems):
    wid = lax.axis_index("s") * NUM_CORES + lax.axis_index("c")
    base = wid * b_per_w
    lag = NBUF - 1

    pltpu.sync_copy(kf_hbm.at[pl.ds(base, b_per_w)],
                    kf_s.at[pl.ds(PAD, b_per_w)])
    pltpu.sync_copy(ord_hbm.at[pl.ds(base, b_per_w)],
                    ord_s.at[pl.ds(PAD, b_per_w)])

    def kf(b):
        return kf_s[pl.ds(b + PAD, LANES)][0]

    def ordv(b):
        return ord_s[pl.ds(b + PAD, LANES)][0]

    def sid(b):
        return kf(b) >> 10

    def fid(b):
        return kf(b) & 1023

    def key(b):
        return kf(b) >> 7

    def slab_src(b):
        off = pl.multiple_of((fid(b) // FBLK) * FBLK, FBLK)
        return lat_hbm.at[sid(b), :, pl.ds(off, FBLK)]

    def mu_src(b):
        return mu_hbm.at[sid(b), :]

    d_iota = lax.broadcasted_iota(jnp.int32, (LANES,), 0)
    one = jnp.int32(1)
    zero = jnp.int32(0)

    def wrap_inc(s):
        return jnp.where(s == NBUF - 1, 0, s + 1)

    def step(j, carry):
        slot_f, slot_e, pk_f, pk_e = carry

        key_j = key(j)
        new_f = jnp.where(
            j < b_per_w,
            jnp.where(j == 0, one, (key_j != pk_f).astype(jnp.int32)),
            zero)
        slot_f2 = jnp.where(new_f == 1, wrap_inc(slot_f), slot_f)
        pk_f2 = jnp.where(j < b_per_w, key_j, pk_f)

        @pl.when(new_f == 1)
        def _fire():
            for k in range(NBUF):
                @pl.when(slot_f2 == k)
                def _(k=k):
                    pltpu.async_copy(slab_src(j),
                                     slabring.at[pl.ds(k * latent_dim,
                                                       latent_dim), :],
                                     sems[k])
                    pltpu.async_copy(mu_src(j), murring.at[k], sems[k])

        e = j - lag
        key_e = key(e)
        new_e = jnp.where(
            j >= lag,
            jnp.where(e == 0, one, (key_e != pk_e).astype(jnp.int32)),
            zero)
        slot_e2 = jnp.where(new_e == 1, wrap_inc(slot_e), slot_e)
        pk_e2 = jnp.where(j >= lag, key_e, pk_e)

        @pl.when(j >= lag)
        def _consume():
            @pl.when(new_e == 1)
            def _wait():
                for k in range(NBUF):
                    @pl.when(slot_e2 == k)
                    def _(k=k):
                        pltpu.make_async_copy(
                            slab_src(e),
                            slabring.at[pl.ds(k * latent_dim,
                                              latent_dim), :],
                            sems[k]).wait()
                        pltpu.make_async_copy(mu_src(e), murring.at[k],
                                              sems[k]).wait()

            @pl.when(e >= RN)
            def _retire():
                for r in range(RN):
                    @pl.when(e % RN == r)
                    def _(r=r):
                        pltpu.make_async_copy(
                            res.at[r], out_hbm.at[ordv(e - RN), :],
                            osems[r]).wait()

            foff = fid(e) % FBLK
            row0 = slot_e2 * latent_dim
            col_idx = jnp.full((LANES,), foff, jnp.int32)
            for d0 in range(0, latent_dim, LANES):
                lat = plsc.load_gather(slabring,
                                       [row0 + d0 + d_iota, col_idx])
                mu = murring[slot_e2, pl.ds(d0, LANES)]
                res[e % RN, pl.ds(d0, LANES)] = mu + SIGMA * (lat - mu)

            for r in range(RN):
                @pl.when(e % RN == r)
                def _(r=r):
                    pltpu.async_copy(res.at[r], out_hbm.at[ordv(e), :],
                                     osems[r])

        return slot_f2, slot_e2, pk_f2, pk_e2

    lax.fori_loop(0, b_per_w + lag, step,
                  (jnp.int32(NBUF - 1), jnp.int32(NBUF - 1),
                   jnp.int32(-1), jnp.int32(-1)))

    def drain(e, _):
        for r in range(RN):
            @pl.when(e % RN == r)
            def _(r=r):
                pltpu.make_async_copy(res.at[r], out_hbm.at[ordv(e), :],
                                      osems[r]).wait()
        return 0

    lax.fori_loop(b_per_w - RN, b_per_w, drain, 0)


def kernel(style_ids, frame_ids, latents, style_latents_mu):
    style_num, frame_num, latent_dim = latents.shape
    batch = style_ids.shape[0]
    b_per_w = batch // NUM_WORKERS
    lat_t = jnp.transpose(latents, (0, 2, 1))

    kf = style_ids * 1024 + frame_ids
    kf_sorted, order = lax.sort_key_val(
        kf, jnp.arange(batch, dtype=jnp.int32))

    mesh = plsc.VectorSubcoreMesh(core_axis_name="c", subcore_axis_name="s",
                                  num_cores=NUM_CORES,
                                  num_subcores=NUM_SUBCORES)
    run = pl.kernel(
        functools.partial(_body, latent_dim, b_per_w),
        out_type=jax.ShapeDtypeStruct((batch, latent_dim), jnp.float32),
        mesh=mesh,
        scratch_types=[
            pltpu.VMEM((b_per_w + PAD + 64,), jnp.int32),
            pltpu.VMEM((b_per_w + PAD + 64,), jnp.int32),
            pltpu.VMEM((NBUF * latent_dim, FBLK), jnp.float32),
            pltpu.VMEM((NBUF, latent_dim), jnp.float32),
            pltpu.VMEM((RN, latent_dim), jnp.float32),
            [pltpu.SemaphoreType.DMA for _ in range(NBUF)],
            [pltpu.SemaphoreType.DMA for _ in range(RN)],
        ],
        compiler_params=pltpu.CompilerParams(needs_layout_passes=False),
    )
    return run(kf_sorted, order, lat_t, style_latents_mu)
